# baseline (speedup 1.0000x reference)
import functools

import jax
import jax.numpy as jnp
from jax import lax
from jax.experimental import pallas as pl
from jax.experimental.pallas import tpu as pltpu
from jax.experimental.pallas import tpu_sc as plsc

_LANES = 16


@jax.jit
def _sc_gather(t_idx, table):
    batch = t_idx.shape[0]
    table_size = table.shape[0]
    info = plsc.get_sparse_core_info()
    num_subcores = 8
    num_workers = num_subcores
    per_worker = batch // num_workers

    mesh = plsc.VectorSubcoreMesh(
        core_axis_name="c", subcore_axis_name="s", num_cores=1,
        num_subcores=num_subcores,
    )

    @functools.partial(
        pl.kernel,
        mesh=mesh,
        out_type=jax.ShapeDtypeStruct((batch,), jnp.float32),
        compiler_params=pltpu.CompilerParams(needs_layout_passes=False),
        scratch_types=[
            pltpu.VMEM((per_worker,), jnp.int32),
            pltpu.VMEM((table_size,), jnp.float32),
            pltpu.VMEM((per_worker,), jnp.float32),
            pltpu.SemaphoreType.DMA,
        ],
    )
    def gather_kernel(t_hbm, table_hbm, out_hbm, idx_v, table_v, out_v, sem):
        wid = lax.axis_index("s")
        base = wid * per_worker
        cp_idx = pltpu.make_async_copy(
            t_hbm.at[pl.ds(base, per_worker)], idx_v, sem
        )
        cp_tab = pltpu.make_async_copy(table_hbm, table_v, sem)
        cp_idx.start()
        cp_tab.start()
        cp_idx.wait()
        cp_tab.wait()

        def body(i, carry):
            off = i * _LANES
            idx_vec = idx_v[pl.ds(off, _LANES)]
            out_v[pl.ds(off, _LANES)] = plsc.load_gather(table_v, [idx_vec])
            return carry

        lax.fori_loop(0, per_worker // _LANES, body, 0)
        pltpu.sync_copy(out_v, out_hbm.at[pl.ds(base, per_worker)])

    return gather_kernel(t_idx, table)


def kernel(t_int, betas):
    return _sc_gather(t_int.astype(jnp.int32), betas)
